# Initial kernel scaffold; baseline (speedup 1.0000x reference)
import jax
import jax.numpy as jnp
from jax.experimental import pallas as pl


def kernel(x, edge_index, batch, W1, b1, W2, b2):
    raise NotImplementedError("write your pallas kernel here")


# speedup vs baseline: 9.1411x; 9.1411x over previous
import functools

import jax
import jax.numpy as jnp
from jax import lax
from jax.experimental import pallas as pl
from jax.experimental.pallas import tpu as pltpu
from jax.experimental.pallas import tpu_sc as plsc

N = 10000
E = 320000
G = 64
D_IN = 128
D_HID = 256
D_OUT = 128

NC = 2
NS = 16
KB = 128
NB = 160
SEC = 16
EPT = NB * KB
EPAD = EPT * NS
NP = 10240
RPT = NP // NS
RB = 512
GRID = NP // RB

_mesh = plsc.VectorSubcoreMesh(
    core_axis_name="c", subcore_axis_name="s", num_cores=NC, num_subcores=NS
)

_DEG_HALF = NB // 2


@functools.partial(
    pl.kernel,
    out_type=jax.ShapeDtypeStruct((NC, NP, 1), jnp.float32),
    mesh=_mesh,
    scratch_types=[
        pltpu.VMEM((SEC, KB), jnp.int32),
        pltpu.VMEM((KB, 1), jnp.float32),
        pltpu.VMEM_SHARED((NP, 1), jnp.float32),
    ],
)
def _deg_kernel(dst_hbm, ones_hbm, zeros_hbm, out_hbm, didx, ones_v, acc_sh):
    c = lax.axis_index("c")
    s = lax.axis_index("s")
    row0 = s * RPT
    pltpu.sync_copy(ones_hbm, ones_v)
    pltpu.sync_copy(zeros_hbm.at[pl.ds(row0, RPT)], acc_sh.at[pl.ds(row0, RPT)])
    plsc.subcore_barrier()

    def sec_body(t, carry):
        pltpu.sync_copy(dst_hbm.at[s].at[pl.ds(c * _DEG_HALF + t * SEC, SEC)], didx)

        def body(j, carry2):
            pltpu.sync_copy(ones_v, acc_sh.at[didx.at[j]], add=True)
            return carry2

        return lax.fori_loop(0, SEC, body, carry)

    lax.fori_loop(0, _DEG_HALF // SEC, sec_body, 0)
    plsc.subcore_barrier()
    pltpu.sync_copy(
        acc_sh.at[pl.ds(row0, RPT)], out_hbm.at[c].at[pl.ds(row0, RPT)]
    )


def _make_agg_kernel(dc):
    @functools.partial(
        pl.kernel,
        out_type=(
            jax.ShapeDtypeStruct((NP, dc), jnp.float32),
            jax.ShapeDtypeStruct((NP, dc), jnp.float32),
        ),
        mesh=_mesh,
        scratch_types=[
            pltpu.VMEM((SEC, KB), jnp.int32),
            pltpu.VMEM((SEC, KB), jnp.int32),
            pltpu.VMEM((KB, dc), jnp.float32),
            pltpu.VMEM_SHARED((NP, dc), jnp.float32),
            pltpu.SemaphoreType.DMA,
        ],
    )
    def agg_kernel(hs_a, hs_b, src_h, dst_h, agg_a, agg_b, sidx, didx, buf, acc, sem):
        c = lax.axis_index("c")
        s = lax.axis_index("s")
        row0 = s * RPT

        def work(hs_x, agg_x):
            pltpu.sync_copy(hs_x.at[pl.ds(row0, RPT)], acc.at[pl.ds(row0, RPT)])
            plsc.subcore_barrier()

            def sec_body(t, carry):
                pltpu.sync_copy(src_h.at[s].at[pl.ds(t * SEC, SEC)], sidx)
                pltpu.sync_copy(dst_h.at[s].at[pl.ds(t * SEC, SEC)], didx)

                def body(j, carry2):
                    pltpu.async_copy(hs_x.at[sidx.at[j]], buf, sem).wait()
                    pltpu.sync_copy(buf, acc.at[didx.at[j]], add=True)
                    return carry2

                return lax.fori_loop(0, SEC, body, carry)

            lax.fori_loop(0, NB // SEC, sec_body, 0)
            plsc.subcore_barrier()
            pltpu.sync_copy(acc.at[pl.ds(row0, RPT)], agg_x.at[pl.ds(row0, RPT)])

        @pl.when(c == 0)
        def _():
            work(hs_a, agg_a)

        @pl.when(c == 1)
        def _():
            work(hs_b, agg_b)

    return agg_kernel


_agg128 = _make_agg_kernel(D_HID // 2)


@functools.partial(
    pl.kernel,
    out_type=(
        jax.ShapeDtypeStruct((NP, D_OUT), jnp.float32),
        jax.ShapeDtypeStruct((NP, D_OUT), jnp.float32),
    ),
    mesh=_mesh,
    scratch_types=[
        pltpu.VMEM((SEC, KB), jnp.int32),
        pltpu.VMEM((SEC, KB), jnp.int32),
        pltpu.VMEM((KB, D_OUT), jnp.float32),
        pltpu.VMEM_SHARED((NP, D_OUT), jnp.float32),
        pltpu.SemaphoreType.DMA,
    ],
)
def _agg2_kernel(hs2, src_h, dst_h, p0, p1, sidx, didx, buf, acc, sem):
    c = lax.axis_index("c")
    s = lax.axis_index("s")
    row0 = s * RPT
    half = NB // 2

    pltpu.sync_copy(hs2.at[pl.ds(row0, RPT)], acc.at[pl.ds(row0, RPT)])
    plsc.subcore_barrier()

    def sec_body(t, carry):
        base = c * half + t * SEC
        pltpu.sync_copy(src_h.at[s].at[pl.ds(base, SEC)], sidx)
        pltpu.sync_copy(dst_h.at[s].at[pl.ds(base, SEC)], didx)

        def body(j, carry2):
            pltpu.async_copy(hs2.at[sidx.at[j]], buf, sem).wait()
            pltpu.sync_copy(buf, acc.at[didx.at[j]], add=True)
            return carry2

        return lax.fori_loop(0, SEC, body, carry)

    lax.fori_loop(0, half // SEC, sec_body, 0)
    plsc.subcore_barrier()

    @pl.when(c == 0)
    def _():
        pltpu.sync_copy(acc.at[pl.ds(row0, RPT)], p0.at[pl.ds(row0, RPT)])

    @pl.when(c == 1)
    def _():
        pltpu.sync_copy(acc.at[pl.ds(row0, RPT)], p1.at[pl.ds(row0, RPT)])


def _dinv_block(degp):
    deg = degp[0] + degp[1] + 1.0
    return lax.rsqrt(deg)


def _b_body(degp_ref, x_ref, w1_ref, hsa_ref, hsb_ref):
    dinv = _dinv_block(degp_ref[...])
    h = jnp.dot(x_ref[...], w1_ref[...], preferred_element_type=jnp.float32)
    hs = h * dinv
    hsa_ref[...] = hs[:, : D_HID // 2]
    hsb_ref[...] = hs[:, D_HID // 2 :]


def _d_body(degp_ref, a1a_ref, a1b_ref, b1_ref, w2_ref, h1_ref, hs2_ref):
    dinv = _dinv_block(degp_ref[...])
    agg = jnp.concatenate([a1a_ref[...], a1b_ref[...]], axis=1)
    h1 = jnp.maximum(agg * dinv + b1_ref[...], 0.0)
    h1_ref[...] = h1
    hs2 = jnp.dot(h1, w2_ref[...], preferred_element_type=jnp.float32) * dinv
    hs2_ref[...] = hs2


def _f_body(degp_ref, p0_ref, p1_ref, hs2_ref, b2_ref, h1_ref, batch_ref,
            h2_ref, xpool_ref, psum, cnt):
    i = pl.program_id(0)
    dinv = _dinv_block(degp_ref[...])
    agg2 = p0_ref[...] + p1_ref[...] - hs2_ref[...]
    h2 = jnp.maximum(agg2 * dinv + b2_ref[...], 0.0)
    h2_ref[...] = h2
    bt = batch_ref[0]
    gids = lax.broadcasted_iota(jnp.int32, (G, RB), 0)
    oh = (gids == bt).astype(jnp.float32)
    cat = jnp.concatenate([h1_ref[...], h2], axis=1)
    ps = jnp.dot(oh, cat, preferred_element_type=jnp.float32)
    cn = jnp.sum(oh, axis=1, keepdims=True)

    @pl.when(i == 0)
    def _():
        psum[...] = ps
        cnt[...] = cn

    @pl.when(i > 0)
    def _():
        psum[...] += ps
        cnt[...] += cn

    xpool_ref[...] = psum[...] / jnp.maximum(cnt[...], 1.0)


def kernel(x, edge_index, batch, W1, b1, W2, b2):
    f32 = jnp.float32
    src = edge_index[0]
    dst = edge_index[1]

    pad = EPT - E // NS
    src3 = jnp.pad(src.reshape(NS, E // NS), ((0, 0), (0, pad))).reshape(NS, NB, KB)
    dst3 = jnp.pad(
        dst.reshape(NS, E // NS), ((0, 0), (0, pad)), constant_values=N
    ).reshape(NS, NB, KB)

    xp = jnp.pad(x, ((0, NP - N), (0, 0)))
    batch3 = jnp.pad(batch.astype(jnp.int32), (0, NP - N), constant_values=G).reshape(
        GRID, 1, RB
    )
    ones_c = jnp.ones((KB, 1), f32)
    zeros_c = jnp.zeros((NP, 1), f32)

    degp = _deg_kernel(dst3, ones_c, zeros_c)

    dc1 = D_HID // 2
    dc2 = D_OUT // 2

    grid = (GRID,)
    degp_spec = pl.BlockSpec((NC, RB, 1), lambda i: (0, i, 0))
    row_spec = lambda w: pl.BlockSpec((RB, w), lambda i: (i, 0))
    full_spec = lambda r, c: pl.BlockSpec((r, c), lambda i: (0, 0))

    hs1a, hs1b = pl.pallas_call(
        _b_body,
        grid=grid,
        in_specs=[degp_spec, row_spec(D_IN), full_spec(D_IN, D_HID)],
        out_specs=[row_spec(dc1), row_spec(dc1)],
        out_shape=[
            jax.ShapeDtypeStruct((NP, dc1), f32),
            jax.ShapeDtypeStruct((NP, dc1), f32),
        ],
    )(degp, xp, W1)

    agg1a, agg1b = _agg128(hs1a, hs1b, src3, dst3)

    h1, hs2 = pl.pallas_call(
        _d_body,
        grid=grid,
        in_specs=[
            degp_spec,
            row_spec(dc1),
            row_spec(dc1),
            full_spec(1, D_HID),
            full_spec(D_HID, D_OUT),
        ],
        out_specs=[row_spec(D_HID), row_spec(D_OUT)],
        out_shape=[
            jax.ShapeDtypeStruct((NP, D_HID), f32),
            jax.ShapeDtypeStruct((NP, D_OUT), f32),
        ],
    )(degp, agg1a, agg1b, b1.reshape(1, D_HID), W2)

    p0, p1 = _agg2_kernel(hs2, src3, dst3)

    h2, xpool = pl.pallas_call(
        _f_body,
        grid=grid,
        in_specs=[
            degp_spec,
            row_spec(D_OUT),
            row_spec(D_OUT),
            row_spec(D_OUT),
            full_spec(1, D_OUT),
            row_spec(D_HID),
            pl.BlockSpec((1, 1, RB), lambda i: (i, 0, 0)),
        ],
        out_specs=[row_spec(D_OUT), pl.BlockSpec((G, D_HID + D_OUT), lambda i: (0, 0))],
        out_shape=[
            jax.ShapeDtypeStruct((NP, D_OUT), f32),
            jax.ShapeDtypeStruct((G, D_HID + D_OUT), f32),
        ],
        scratch_shapes=[
            pltpu.VMEM((G, D_HID + D_OUT), f32),
            pltpu.VMEM((G, 1), f32),
        ],
    )(degp, p0, p1, hs2, b2.reshape(1, D_OUT), h1, batch3)

    xcat = jnp.concatenate([h1[:N], h2[:N]], axis=1)
    return (xpool, xcat)


# speedup vs baseline: 10.1387x; 1.1091x over previous
import functools

import jax
import jax.numpy as jnp
from jax import lax
from jax.experimental import pallas as pl
from jax.experimental.pallas import tpu as pltpu
from jax.experimental.pallas import tpu_sc as plsc

N = 10000
E = 320000
G = 64
D_IN = 128
D_HID = 256
D_OUT = 128

NC = 2
NS = 16
KB = 128
NB = 160
SEC = 16
EPT = NB * KB
EPAD = EPT * NS
NP = 10240
RPT = NP // NS
RB = 512
GRID = NP // RB

_mesh = plsc.VectorSubcoreMesh(
    core_axis_name="c", subcore_axis_name="s", num_cores=NC, num_subcores=NS
)

_DEG_HALF = NB // 2


@functools.partial(
    pl.kernel,
    out_type=jax.ShapeDtypeStruct((NC, NP, 1), jnp.float32),
    mesh=_mesh,
    scratch_types=[
        pltpu.VMEM((SEC, KB), jnp.int32),
        pltpu.VMEM((KB, 1), jnp.float32),
        pltpu.VMEM_SHARED((NP, 1), jnp.float32),
    ],
)
def _deg_kernel(dst_hbm, ones_hbm, zeros_hbm, out_hbm, didx, ones_v, acc_sh):
    c = lax.axis_index("c")
    s = lax.axis_index("s")
    row0 = s * RPT
    pltpu.sync_copy(ones_hbm, ones_v)
    pltpu.sync_copy(zeros_hbm.at[pl.ds(row0, RPT)], acc_sh.at[pl.ds(row0, RPT)])
    plsc.subcore_barrier()

    def sec_body(t, carry):
        pltpu.sync_copy(dst_hbm.at[s].at[pl.ds(c * _DEG_HALF + t * SEC, SEC)], didx)

        def body(j, carry2):
            pltpu.sync_copy(ones_v, acc_sh.at[didx.at[j]], add=True)
            return carry2

        return lax.fori_loop(0, SEC, body, carry)

    lax.fori_loop(0, _DEG_HALF // SEC, sec_body, 0)
    plsc.subcore_barrier()
    pltpu.sync_copy(
        acc_sh.at[pl.ds(row0, RPT)], out_hbm.at[c].at[pl.ds(row0, RPT)]
    )


def _make_agg_kernel(dc):
    @functools.partial(
        pl.kernel,
        out_type=(
            jax.ShapeDtypeStruct((NP, dc), jnp.float32),
            jax.ShapeDtypeStruct((NP, dc), jnp.float32),
        ),
        mesh=_mesh,
        scratch_types=[
            pltpu.VMEM((SEC, KB), jnp.int32),
            pltpu.VMEM((SEC, KB), jnp.int32),
            pltpu.VMEM((KB, dc), jnp.float32),
            pltpu.VMEM((KB, dc), jnp.float32),
            pltpu.VMEM_SHARED((NP, dc), jnp.float32),
            pltpu.SemaphoreType.DMA,
            pltpu.SemaphoreType.DMA,
        ],
    )
    def agg_kernel(hs_a, hs_b, src_h, dst_h, agg_a, agg_b,
                   sidx, didx, buf0, buf1, acc, sem0, sem1):
        c = lax.axis_index("c")
        s = lax.axis_index("s")
        row0 = s * RPT

        def work(hs_x, agg_x):
            pltpu.sync_copy(hs_x.at[pl.ds(row0, RPT)], acc.at[pl.ds(row0, RPT)])
            plsc.subcore_barrier()

            def sec_body(t, carry):
                pltpu.sync_copy(src_h.at[s].at[pl.ds(t * SEC, SEC)], sidx)
                pltpu.sync_copy(dst_h.at[s].at[pl.ds(t * SEC, SEC)], didx)
                pltpu.async_copy(hs_x.at[sidx.at[0]], buf0, sem0)

                def pair(j2, carry2):
                    j = j2 * 2
                    pltpu.make_async_copy(hs_x.at[sidx.at[j]], buf0, sem0).wait()
                    pltpu.async_copy(hs_x.at[sidx.at[j + 1]], buf1, sem1)
                    pltpu.sync_copy(buf0, acc.at[didx.at[j]], add=True)
                    pltpu.make_async_copy(
                        hs_x.at[sidx.at[j + 1]], buf1, sem1
                    ).wait()

                    @pl.when(j2 < SEC // 2 - 1)
                    def _():
                        pltpu.async_copy(hs_x.at[sidx.at[j + 2]], buf0, sem0)

                    pltpu.sync_copy(buf1, acc.at[didx.at[j + 1]], add=True)
                    return carry2

                return lax.fori_loop(0, SEC // 2, pair, carry)

            lax.fori_loop(0, NB // SEC, sec_body, 0)
            plsc.subcore_barrier()
            pltpu.sync_copy(acc.at[pl.ds(row0, RPT)], agg_x.at[pl.ds(row0, RPT)])

        @pl.when(c == 0)
        def _():
            work(hs_a, agg_a)

        @pl.when(c == 1)
        def _():
            work(hs_b, agg_b)

    return agg_kernel


_agg128 = _make_agg_kernel(D_HID // 2)


@functools.partial(
    pl.kernel,
    out_type=(
        jax.ShapeDtypeStruct((NP, D_OUT), jnp.float32),
        jax.ShapeDtypeStruct((NP, D_OUT), jnp.float32),
    ),
    mesh=_mesh,
    scratch_types=[
        pltpu.VMEM((SEC, KB), jnp.int32),
        pltpu.VMEM((SEC, KB), jnp.int32),
        pltpu.VMEM((KB, D_OUT), jnp.float32),
        pltpu.VMEM((KB, D_OUT), jnp.float32),
        pltpu.VMEM_SHARED((NP, D_OUT), jnp.float32),
        pltpu.SemaphoreType.DMA,
        pltpu.SemaphoreType.DMA,
    ],
)
def _agg2_kernel(hs2, src_h, dst_h, p0, p1, sidx, didx, buf0, buf1, acc, sem0, sem1):
    c = lax.axis_index("c")
    s = lax.axis_index("s")
    row0 = s * RPT
    half = NB // 2

    pltpu.sync_copy(hs2.at[pl.ds(row0, RPT)], acc.at[pl.ds(row0, RPT)])
    plsc.subcore_barrier()

    def sec_body(t, carry):
        base = c * half + t * SEC
        pltpu.sync_copy(src_h.at[s].at[pl.ds(base, SEC)], sidx)
        pltpu.sync_copy(dst_h.at[s].at[pl.ds(base, SEC)], didx)
        pltpu.async_copy(hs2.at[sidx.at[0]], buf0, sem0)

        def pair(j2, carry2):
            j = j2 * 2
            pltpu.make_async_copy(hs2.at[sidx.at[j]], buf0, sem0).wait()
            pltpu.async_copy(hs2.at[sidx.at[j + 1]], buf1, sem1)
            pltpu.sync_copy(buf0, acc.at[didx.at[j]], add=True)
            pltpu.make_async_copy(hs2.at[sidx.at[j + 1]], buf1, sem1).wait()

            @pl.when(j2 < SEC // 2 - 1)
            def _():
                pltpu.async_copy(hs2.at[sidx.at[j + 2]], buf0, sem0)

            pltpu.sync_copy(buf1, acc.at[didx.at[j + 1]], add=True)
            return carry2

        return lax.fori_loop(0, SEC // 2, pair, carry)

    lax.fori_loop(0, half // SEC, sec_body, 0)
    plsc.subcore_barrier()

    @pl.when(c == 0)
    def _():
        pltpu.sync_copy(acc.at[pl.ds(row0, RPT)], p0.at[pl.ds(row0, RPT)])

    @pl.when(c == 1)
    def _():
        pltpu.sync_copy(acc.at[pl.ds(row0, RPT)], p1.at[pl.ds(row0, RPT)])


def _dinv_block(degp):
    deg = degp[0] + degp[1] + 1.0
    return lax.rsqrt(deg)


def _b_body(degp_ref, x_ref, w1_ref, hsa_ref, hsb_ref):
    dinv = _dinv_block(degp_ref[...])
    h = jnp.dot(x_ref[...], w1_ref[...], preferred_element_type=jnp.float32)
    hs = h * dinv
    hsa_ref[...] = hs[:, : D_HID // 2]
    hsb_ref[...] = hs[:, D_HID // 2 :]


def _d_body(degp_ref, a1a_ref, a1b_ref, b1_ref, w2_ref, h1_ref, hs2_ref):
    dinv = _dinv_block(degp_ref[...])
    agg = jnp.concatenate([a1a_ref[...], a1b_ref[...]], axis=1)
    h1 = jnp.maximum(agg * dinv + b1_ref[...], 0.0)
    h1_ref[...] = h1
    hs2 = jnp.dot(h1, w2_ref[...], preferred_element_type=jnp.float32) * dinv
    hs2_ref[...] = hs2


def _f_body(degp_ref, p0_ref, p1_ref, hs2_ref, b2_ref, h1_ref, batch_ref,
            h2_ref, xpool_ref, psum, cnt):
    i = pl.program_id(0)
    dinv = _dinv_block(degp_ref[...])
    agg2 = p0_ref[...] + p1_ref[...] - hs2_ref[...]
    h2 = jnp.maximum(agg2 * dinv + b2_ref[...], 0.0)
    h2_ref[...] = h2
    bt = batch_ref[0]
    gids = lax.broadcasted_iota(jnp.int32, (G, RB), 0)
    oh = (gids == bt).astype(jnp.float32)
    cat = jnp.concatenate([h1_ref[...], h2], axis=1)
    ps = jnp.dot(oh, cat, preferred_element_type=jnp.float32)
    cn = jnp.sum(oh, axis=1, keepdims=True)

    @pl.when(i == 0)
    def _():
        psum[...] = ps
        cnt[...] = cn

    @pl.when(i > 0)
    def _():
        psum[...] += ps
        cnt[...] += cn

    xpool_ref[...] = psum[...] / jnp.maximum(cnt[...], 1.0)


def kernel(x, edge_index, batch, W1, b1, W2, b2):
    f32 = jnp.float32
    src = edge_index[0]
    dst = edge_index[1]

    pad = EPT - E // NS
    src3 = jnp.pad(src.reshape(NS, E // NS), ((0, 0), (0, pad))).reshape(NS, NB, KB)
    dst3 = jnp.pad(
        dst.reshape(NS, E // NS), ((0, 0), (0, pad)), constant_values=N
    ).reshape(NS, NB, KB)

    xp = jnp.pad(x, ((0, NP - N), (0, 0)))
    batch3 = jnp.pad(batch.astype(jnp.int32), (0, NP - N), constant_values=G).reshape(
        GRID, 1, RB
    )
    ones_c = jnp.ones((KB, 1), f32)
    zeros_c = jnp.zeros((NP, 1), f32)

    degp = _deg_kernel(dst3, ones_c, zeros_c)

    dc1 = D_HID // 2
    dc2 = D_OUT // 2

    grid = (GRID,)
    degp_spec = pl.BlockSpec((NC, RB, 1), lambda i: (0, i, 0))
    row_spec = lambda w: pl.BlockSpec((RB, w), lambda i: (i, 0))
    full_spec = lambda r, c: pl.BlockSpec((r, c), lambda i: (0, 0))

    hs1a, hs1b = pl.pallas_call(
        _b_body,
        grid=grid,
        in_specs=[degp_spec, row_spec(D_IN), full_spec(D_IN, D_HID)],
        out_specs=[row_spec(dc1), row_spec(dc1)],
        out_shape=[
            jax.ShapeDtypeStruct((NP, dc1), f32),
            jax.ShapeDtypeStruct((NP, dc1), f32),
        ],
    )(degp, xp, W1)

    agg1a, agg1b = _agg128(hs1a, hs1b, src3, dst3)

    h1, hs2 = pl.pallas_call(
        _d_body,
        grid=grid,
        in_specs=[
            degp_spec,
            row_spec(dc1),
            row_spec(dc1),
            full_spec(1, D_HID),
            full_spec(D_HID, D_OUT),
        ],
        out_specs=[row_spec(D_HID), row_spec(D_OUT)],
        out_shape=[
            jax.ShapeDtypeStruct((NP, D_HID), f32),
            jax.ShapeDtypeStruct((NP, D_OUT), f32),
        ],
    )(degp, agg1a, agg1b, b1.reshape(1, D_HID), W2)

    p0, p1 = _agg2_kernel(hs2, src3, dst3)

    h2, xpool = pl.pallas_call(
        _f_body,
        grid=grid,
        in_specs=[
            degp_spec,
            row_spec(D_OUT),
            row_spec(D_OUT),
            row_spec(D_OUT),
            full_spec(1, D_OUT),
            row_spec(D_HID),
            pl.BlockSpec((1, 1, RB), lambda i: (i, 0, 0)),
        ],
        out_specs=[row_spec(D_OUT), pl.BlockSpec((G, D_HID + D_OUT), lambda i: (0, 0))],
        out_shape=[
            jax.ShapeDtypeStruct((NP, D_OUT), f32),
            jax.ShapeDtypeStruct((G, D_HID + D_OUT), f32),
        ],
        scratch_shapes=[
            pltpu.VMEM((G, D_HID + D_OUT), f32),
            pltpu.VMEM((G, 1), f32),
        ],
    )(degp, p0, p1, hs2, b2.reshape(1, D_OUT), h1, batch3)

    xcat = jnp.concatenate([h1[:N], h2[:N]], axis=1)
    return (xpool, xcat)
